# baseline (speedup 1.0000x reference)
import functools

import jax
import jax.numpy as jnp
from jax import lax
from jax.experimental import pallas as pl
from jax.experimental.pallas import tpu as pltpu
from jax.experimental.pallas import tpu_sc as plsc

_NC = 2
_NS = 16
_L = 16
_NW = _NC * _NS


@functools.partial(jax.jit, static_argnames=("n_chunks", "k_rows", "d_model"))
def _sc_gather(idx3, table, *, n_chunks, k_rows, d_model):
    b_total = _NW * n_chunks * k_rows
    max_row = table.shape[0] - 1
    mesh = plsc.VectorSubcoreMesh(core_axis_name="c", subcore_axis_name="s")
    assert n_chunks >= 5 and (n_chunks - 5) % 3 == 0

    def body(idx_hbm, tbl_hbm, out_hbm, idx_v,
             buf0, buf1, buf2, gsem0, gsem1, gsem2, osem0, osem1, osem2):
        wid = lax.axis_index("s") * _NC + lax.axis_index("c")
        base = wid * (n_chunks * k_rows)

        pltpu.sync_copy(idx_hbm.at[wid], idx_v)

        @pl.loop(0, n_chunks)
        def _clamp(c):
            v = idx_v[c]
            idx_v[c] = jnp.where(v == jnp.int32(-1), jnp.int32(max_row), v)

        bufs = (buf0, buf1, buf2)
        gsems = (gsem0, gsem1, gsem2)
        osems = (osem0, osem1, osem2)

        def wait_gather(b):
            pltpu.make_async_copy(tbl_hbm.at[idx_v.at[0]], bufs[b], gsems[b]).wait()

        def wait_out(b):
            pltpu.make_async_copy(
                bufs[b], out_hbm.at[pl.ds(base, k_rows)], osems[b]
            ).wait()

        def start_gather(cc, b):
            pltpu.async_copy(tbl_hbm.at[idx_v.at[cc]], bufs[b], gsems[b])

        def start_out(cc, b):
            pltpu.async_copy(
                bufs[b], out_hbm.at[pl.ds(base + cc * k_rows, k_rows)], osems[b]
            )

        start_gather(0, 0)
        start_gather(1, 1)

        wait_gather(0)
        start_out(0, 0)
        start_gather(2, 2)

        wait_gather(1)
        start_out(1, 1)
        wait_out(0)
        start_gather(3, 0)

        wait_gather(2)
        start_out(2, 2)
        wait_out(1)
        start_gather(4, 1)

        @pl.loop(3, n_chunks - 2, step=3)
        def _main(c):
            for j in range(3):
                cc = c + j
                b = j
                wait_gather(b)
                start_out(cc, b)
                nb = (j + 2) % 3
                wait_out(nb)
                start_gather(cc + 2, nb)

        for cc in (n_chunks - 2, n_chunks - 1):
            b = cc % 3
            wait_gather(b)
            start_out(cc, b)
        for b in range(3):
            wait_out(b)

    run = pl.kernel(
        body,
        out_type=jax.ShapeDtypeStruct((b_total, d_model), jnp.float32),
        mesh=mesh,
        scratch_types=[
            pltpu.VMEM((n_chunks, _L), jnp.int32),
            pltpu.VMEM((k_rows, d_model), jnp.float32),
            pltpu.VMEM((k_rows, d_model), jnp.float32),
            pltpu.VMEM((k_rows, d_model), jnp.float32),
            pltpu.SemaphoreType.DMA,
            pltpu.SemaphoreType.DMA,
            pltpu.SemaphoreType.DMA,
            pltpu.SemaphoreType.DMA,
            pltpu.SemaphoreType.DMA,
            pltpu.SemaphoreType.DMA,
        ],
    )
    return run(idx3, table)


def kernel(indices, pos_encodings):
    d_model = pos_encodings.shape[1]
    b_total = indices.size
    k_rows = _L
    n_chunks = b_total // (_NW * k_rows)
    idx3 = indices.reshape(_NW, n_chunks, k_rows)
    out = _sc_gather(idx3, pos_encodings, n_chunks=n_chunks, k_rows=k_rows, d_model=d_model)
    return out.reshape(indices.shape + (d_model,))
